# baseline (speedup 1.0000x reference)
import functools

import jax
import jax.numpy as jnp
from jax import lax
from jax.experimental import pallas as pl
from jax.experimental.pallas import tpu as pltpu
from jax.experimental.pallas import tpu_sc as plsc

_GW = 128


def _sc_gather(sec_es, fst16, idx, f, e, b_sz):
    n = idx.shape[0]
    nb = b_sz // _GW
    mesh = plsc.VectorSubcoreMesh(core_axis_name="c", subcore_axis_name="s")

    @functools.partial(
        pl.kernel,
        out_type=[
            jax.ShapeDtypeStruct((f, e, b_sz), jnp.float32),
            jax.ShapeDtypeStruct((n,), jnp.float32),
        ],
        mesh=mesh,
        scratch_types=(
            [pltpu.VMEM((_GW,), jnp.int32)]
            + [pltpu.VMEM((_GW, 16), jnp.float32) for _ in range(e + 1)]
            + [pltpu.SemaphoreType.DMA]
        ),
        compiler_params=pltpu.CompilerParams(use_tc_tiling_on_sc=False,
                                             needs_layout_passes=False),
    )
    def k(*args):
        tabs = args[:e]
        fst_hbm = args[e]
        i_hbm = args[e + 1]
        osec_hbm, ofst_hbm = args[e + 2], args[e + 3]
        fridx_v = args[e + 4]
        rows_v = args[e + 5:e + 5 + e]
        frows_v = args[2 * e + 5]
        sem = args[2 * e + 6]

        def body(i_vmem, osec_vmem, ofst_vmem):
            @pl.loop(0, _GW, step=16)
            def _(c):
                fridx_v[pl.ds(c, 16)] = lax.shift_right_logical(
                    i_vmem[pl.ds(c, 16)], 4)

            copies = [pltpu.async_copy(tabs[j].at[fridx_v], rows_v[j], sem)
                      for j in range(e)]
            copies.append(pltpu.async_copy(fst_hbm.at[fridx_v], frows_v, sem))
            for cp in copies:
                cp.wait()
            lane16 = lax.iota(jnp.int32, 16)

            @pl.loop(0, _GW, step=16)
            def _(c):
                rows = lane16 + c
                lane = jnp.bitwise_and(i_vmem[pl.ds(c, 16)], 15)
                ofst_vmem[pl.ds(c, 16)] = plsc.load_gather(frows_v,
                                                           [rows, lane])
                for j in range(e):
                    osec_vmem[0, j, pl.ds(c, 16)] = plsc.load_gather(
                        rows_v[j], [rows, lane])

        pltpu.emit_pipeline(
            body,
            grid=(n // _GW,),
            in_specs=[pl.BlockSpec((_GW,), lambda i: (i,))],
            out_specs=[
                pl.BlockSpec((1, e, _GW), lambda i: (i // nb, 0, i % nb)),
                pl.BlockSpec((_GW,), lambda i: (i,)),
            ],
            core_axis_name=("c", "s"),
            dimension_semantics=(pltpu.PARALLEL,),
        )(i_hbm, osec_hbm, ofst_hbm)

    return k(*sec_es, fst16, idx)


def _tc_block(f, e, sec_ref, fst_ref, xv_ref, w1t_ref, b1_ref, w2t_ref,
              b2_ref, bias_ref, out_ref):
    hi = lax.Precision.HIGHEST
    sec_raw = sec_ref[...]
    xv = xv_ref[...]

    li = lax.broadcasted_iota(jnp.int32, (f * e, f), 0)
    fi = lax.broadcasted_iota(jnp.int32, (f * e, f), 1)
    erep = (li // e == fi).astype(jnp.float32)
    sec = sec_raw * jnp.dot(erep, xv, precision=hi)

    g1 = lax.broadcasted_iota(jnp.int32, (e, f * e), 0)
    g2 = lax.broadcasted_iota(jnp.int32, (e, f * e), 1)
    grep = (g2 % e == g1).astype(jnp.float32)
    s1 = jnp.dot(grep, sec, precision=hi)
    s2 = jnp.dot(grep, sec * sec, precision=hi)
    fm = 0.5 * (s1 * s1 - s2)

    h = jnp.maximum(jnp.dot(w1t_ref[...], sec, precision=hi) + b1_ref[...], 0.0)
    d = jnp.maximum(jnp.dot(w2t_ref[...], h, precision=hi) + b2_ref[...], 0.0)

    fst_sum = jnp.sum(fst_ref[...] * xv, axis=0, keepdims=True)
    out_ref[...] = (fst_sum + jnp.sum(fm, axis=0, keepdims=True)
                    + jnp.sum(d, axis=0, keepdims=True) + bias_ref[...])


def kernel(Xi, Xv, fst_tables, sec_tables, W1, b1, W2, b2, bias):
    b_sz, f, _ = Xi.shape
    v = sec_tables.shape[1]
    e = sec_tables.shape[2]
    h1 = W1.shape[1]
    h2 = W2.shape[1]

    sec_t3 = jnp.transpose(sec_tables, (0, 2, 1))
    sec_es = [sec_t3[:, j, :].reshape(f * v // 16, 16) for j in range(e)]
    fst16 = fst_tables.reshape(f * v // 16, 16)

    xi_t = Xi[:, :, 0].astype(jnp.int32).T
    offs = jnp.arange(f, dtype=jnp.int32)[:, None]
    idx = (xi_t + offs * v).reshape(b_sz * f)

    sec_g, fst_g = _sc_gather(sec_es, fst16, idx, f, e, b_sz)
    sec_g = sec_g.reshape(f * e, b_sz)
    fst_g = fst_g.reshape(f, b_sz)
    xvt = Xv[:, :, 0].T

    bt = 2048
    out = pl.pallas_call(
        functools.partial(_tc_block, f, e),
        grid=(b_sz // bt,),
        in_specs=[
            pl.BlockSpec((f * e, bt), lambda i: (0, i)),
            pl.BlockSpec((f, bt), lambda i: (0, i)),
            pl.BlockSpec((f, bt), lambda i: (0, i)),
            pl.BlockSpec((h1, f * e), lambda i: (0, 0)),
            pl.BlockSpec((h1, 1), lambda i: (0, 0)),
            pl.BlockSpec((h2, h1), lambda i: (0, 0)),
            pl.BlockSpec((h2, 1), lambda i: (0, 0)),
            pl.BlockSpec((1, 1), lambda i: (0, 0)),
        ],
        out_specs=pl.BlockSpec((1, bt), lambda i: (0, i)),
        out_shape=jax.ShapeDtypeStruct((1, b_sz), jnp.float32),
    )(sec_g, fst_g, xvt, W1.T, b1.reshape(h1, 1), W2.T, b2.reshape(h2, 1),
      bias.reshape(1, 1))
    return out.reshape(b_sz)


# speedup vs baseline: 2.0342x; 2.0342x over previous
import functools

import jax
import jax.numpy as jnp
from jax import lax
from jax.experimental import pallas as pl
from jax.experimental.pallas import tpu as pltpu
from jax.experimental.pallas import tpu_sc as plsc

_GW = 128
_VP = 100352


def _split_repack(sec_t3, fst_t2, f, v, e):
    rw = _VP // 8
    tail = v - 7 * rw

    def body(x_ref, fst_ref, *o_refs):
        for j in range(e + 1):
            src = x_ref if j < e else fst_ref
            jj = j if j < e else 0
            for s in range(8):
                w = rw if s < 7 else tail
                o_refs[j][0, s, pl.ds(0, w)] = src[0, jj, pl.ds(s * rw, w)]

    return pl.pallas_call(
        body,
        grid=(f,),
        in_specs=[pl.BlockSpec((1, e, v), lambda i: (i, 0, 0)),
                  pl.BlockSpec((1, 1, v), lambda i: (i, 0, 0))],
        out_specs=[pl.BlockSpec((1, 8, rw), lambda i: (i, 0, 0))
                   for _ in range(e + 1)],
        out_shape=[jax.ShapeDtypeStruct((f, 8, rw), jnp.float32)
                   for _ in range(e + 1)],
    )(sec_t3, fst_t2)


def _sc_gather(sec_es, fst16, idx, f, e, b_sz):
    n = idx.shape[0]
    nb = b_sz // _GW
    mesh = plsc.VectorSubcoreMesh(core_axis_name="c", subcore_axis_name="s")

    @functools.partial(
        pl.kernel,
        out_type=[
            jax.ShapeDtypeStruct((f, e, b_sz), jnp.float32),
            jax.ShapeDtypeStruct((n,), jnp.float32),
        ],
        mesh=mesh,
        scratch_types=(
            [pltpu.VMEM((_GW,), jnp.int32)]
            + [pltpu.VMEM((_GW, 16), jnp.float32) for _ in range(e + 1)]
            + [pltpu.SemaphoreType.DMA]
        ),
        compiler_params=pltpu.CompilerParams(use_tc_tiling_on_sc=False,
                                             needs_layout_passes=False),
    )
    def k(*args):
        tabs = args[:e]
        fst_hbm = args[e]
        i_hbm = args[e + 1]
        osec_hbm, ofst_hbm = args[e + 2], args[e + 3]
        fridx_v = args[e + 4]
        rows_v = args[e + 5:e + 5 + e]
        frows_v = args[2 * e + 5]
        sem = args[2 * e + 6]

        def body(i_vmem, osec_vmem, ofst_vmem):
            @pl.loop(0, _GW, step=16)
            def _(c):
                fridx_v[pl.ds(c, 16)] = lax.shift_right_logical(
                    i_vmem[pl.ds(c, 16)], 4)

            copies = [pltpu.async_copy(tabs[j].at[fridx_v], rows_v[j], sem)
                      for j in range(e)]
            copies.append(pltpu.async_copy(fst_hbm.at[fridx_v], frows_v, sem))
            for cp in copies:
                cp.wait()
            lane16 = lax.iota(jnp.int32, 16)

            @pl.loop(0, _GW, step=16)
            def _(c):
                rows = lane16 + c
                lane = jnp.bitwise_and(i_vmem[pl.ds(c, 16)], 15)
                ofst_vmem[pl.ds(c, 16)] = plsc.load_gather(frows_v,
                                                           [rows, lane])
                for j in range(e):
                    osec_vmem[0, j, pl.ds(c, 16)] = plsc.load_gather(
                        rows_v[j], [rows, lane])

        pltpu.emit_pipeline(
            body,
            grid=(n // _GW,),
            in_specs=[pl.BlockSpec((_GW,), lambda i: (i,))],
            out_specs=[
                pl.BlockSpec((1, e, _GW), lambda i: (i // nb, 0, i % nb)),
                pl.BlockSpec((_GW,), lambda i: (i,)),
            ],
            core_axis_name=("c", "s"),
            dimension_semantics=(pltpu.PARALLEL,),
        )(i_hbm, osec_hbm, ofst_hbm)

    return k(*sec_es, fst16, idx)


def _tc_block(f, e, sec_ref, fst_ref, xv_ref, w1t_ref, b1_ref, w2t_ref,
              b2_ref, bias_ref, out_ref):
    hi = lax.Precision.HIGHEST
    sec_raw = sec_ref[...]
    xv = xv_ref[...]

    li = lax.broadcasted_iota(jnp.int32, (f * e, f), 0)
    fi = lax.broadcasted_iota(jnp.int32, (f * e, f), 1)
    erep = (li // e == fi).astype(jnp.float32)
    sec = sec_raw * jnp.dot(erep, xv, precision=hi)

    g1 = lax.broadcasted_iota(jnp.int32, (e, f * e), 0)
    g2 = lax.broadcasted_iota(jnp.int32, (e, f * e), 1)
    grep = (g2 % e == g1).astype(jnp.float32)
    s1 = jnp.dot(grep, sec, precision=hi)
    s2 = jnp.dot(grep, sec * sec, precision=hi)
    fm = 0.5 * (s1 * s1 - s2)

    h = jnp.maximum(jnp.dot(w1t_ref[...], sec, precision=hi) + b1_ref[...], 0.0)
    d = jnp.maximum(jnp.dot(w2t_ref[...], h, precision=hi) + b2_ref[...], 0.0)

    fst_sum = jnp.sum(fst_ref[...] * xv, axis=0, keepdims=True)
    out_ref[...] = (fst_sum + jnp.sum(fm, axis=0, keepdims=True)
                    + jnp.sum(d, axis=0, keepdims=True) + bias_ref[...])


def kernel(Xi, Xv, fst_tables, sec_tables, W1, b1, W2, b2, bias):
    b_sz, f, _ = Xi.shape
    v = sec_tables.shape[1]
    e = sec_tables.shape[2]
    h1 = W1.shape[1]
    h2 = W2.shape[1]

    sec_t3 = jnp.transpose(sec_tables, (0, 2, 1))
    fst_t2 = jnp.transpose(fst_tables, (0, 2, 1))
    tabs = _split_repack(sec_t3, fst_t2, f, v, e)
    nrow = f * _VP // 16
    sec_es = [t.reshape(nrow, 16) for t in tabs[:e]]
    fst16 = tabs[e].reshape(nrow, 16)

    xi_t = Xi[:, :, 0].astype(jnp.int32).T
    offs = jnp.arange(f, dtype=jnp.int32)[:, None]
    idx = (xi_t + offs * _VP).reshape(b_sz * f)

    sec_g, fst_g = _sc_gather(sec_es, fst16, idx, f, e, b_sz)
    sec_g = sec_g.reshape(f * e, b_sz)
    fst_g = fst_g.reshape(f, b_sz)
    xvt = Xv[:, :, 0].T

    bt = 2048
    out = pl.pallas_call(
        functools.partial(_tc_block, f, e),
        grid=(b_sz // bt,),
        in_specs=[
            pl.BlockSpec((f * e, bt), lambda i: (0, i)),
            pl.BlockSpec((f, bt), lambda i: (0, i)),
            pl.BlockSpec((f, bt), lambda i: (0, i)),
            pl.BlockSpec((h1, f * e), lambda i: (0, 0)),
            pl.BlockSpec((h1, 1), lambda i: (0, 0)),
            pl.BlockSpec((h2, h1), lambda i: (0, 0)),
            pl.BlockSpec((h2, 1), lambda i: (0, 0)),
            pl.BlockSpec((1, 1), lambda i: (0, 0)),
        ],
        out_specs=pl.BlockSpec((1, bt), lambda i: (0, i)),
        out_shape=jax.ShapeDtypeStruct((1, b_sz), jnp.float32),
    )(sec_g, fst_g, xvt, W1.T, b1.reshape(h1, 1), W2.T, b2.reshape(h2, 1),
      bias.reshape(1, 1))
    return out.reshape(b_sz)


# speedup vs baseline: 2.2071x; 1.0850x over previous
import functools

import jax
import jax.numpy as jnp
from jax import lax
from jax.experimental import pallas as pl
from jax.experimental.pallas import tpu as pltpu
from jax.experimental.pallas import tpu_sc as plsc

_GW = 128
_VP = 100352


def _split_repack(sec_t3, fst_t2, f, v, e):
    rw = _VP // 8
    tail = v - 7 * rw

    def body(x_ref, fst_ref, *o_refs):
        for j in range(e + 1):
            src = x_ref if j < e else fst_ref
            jj = j if j < e else 0
            for s in range(8):
                w = rw if s < 7 else tail
                o_refs[j][0, s, pl.ds(0, w)] = src[0, jj, pl.ds(s * rw, w)]

    return pl.pallas_call(
        body,
        grid=(f,),
        in_specs=[pl.BlockSpec((1, e, v), lambda i: (i, 0, 0)),
                  pl.BlockSpec((1, 1, v), lambda i: (i, 0, 0))],
        out_specs=[pl.BlockSpec((1, 8, rw), lambda i: (i, 0, 0))
                   for _ in range(e + 1)],
        out_shape=[jax.ShapeDtypeStruct((f, 8, rw), jnp.float32)
                   for _ in range(e + 1)],
    )(sec_t3, fst_t2)


def _sc_gather(sec_es, fst16, idx, f, e, b_sz):
    n = idx.shape[0]
    nb = b_sz // _GW
    mesh = plsc.VectorSubcoreMesh(core_axis_name="c", subcore_axis_name="s")

    @functools.partial(
        pl.kernel,
        out_type=[
            jax.ShapeDtypeStruct((f, e, b_sz), jnp.float32),
            jax.ShapeDtypeStruct((n,), jnp.float32),
        ],
        mesh=mesh,
        scratch_types=(
            [pltpu.VMEM((_GW,), jnp.int32), pltpu.VMEM((_GW,), jnp.int32)]
            + [pltpu.VMEM((_GW, 16), jnp.float32) for _ in range(e + 1)]
            + [pltpu.SemaphoreType.DMA for _ in range(e + 1)]
        ),
        compiler_params=pltpu.CompilerParams(use_tc_tiling_on_sc=False,
                                             needs_layout_passes=False),
    )
    def k(*args):
        tabs = args[:e]
        fst_hbm = args[e]
        i_hbm = args[e + 1]
        osec_hbm, ofst_hbm = args[e + 2], args[e + 3]
        fridx_v = args[e + 4]
        lane_v = args[e + 5]
        rows_v = args[e + 6:e + 6 + e]
        frows_v = args[2 * e + 6]
        sems = args[2 * e + 7:]

        def body(i_vmem, osec_vmem, ofst_vmem):
            @pl.loop(0, _GW, step=16)
            def _(c):
                iv = i_vmem[pl.ds(c, 16)]
                fridx_v[pl.ds(c, 16)] = lax.shift_right_logical(iv, 4)
                lane_v[pl.ds(c, 16)] = jnp.bitwise_and(iv, 15)

            copies = [pltpu.async_copy(tabs[j].at[fridx_v], rows_v[j], sems[j])
                      for j in range(e)]
            copies.append(pltpu.async_copy(fst_hbm.at[fridx_v], frows_v,
                                           sems[e]))
            lane16 = lax.iota(jnp.int32, 16)

            for j in range(e):
                copies[j].wait()

                @pl.loop(0, _GW, step=16)
                def _(c):
                    osec_vmem[0, j, pl.ds(c, 16)] = plsc.load_gather(
                        rows_v[j], [lane16 + c, lane_v[pl.ds(c, 16)]])

            copies[e].wait()

            @pl.loop(0, _GW, step=16)
            def _(c):
                ofst_vmem[pl.ds(c, 16)] = plsc.load_gather(
                    frows_v, [lane16 + c, lane_v[pl.ds(c, 16)]])

        pltpu.emit_pipeline(
            body,
            grid=(n // _GW,),
            in_specs=[pl.BlockSpec((_GW,), lambda i: (i,))],
            out_specs=[
                pl.BlockSpec((1, e, _GW), lambda i: (i // nb, 0, i % nb)),
                pl.BlockSpec((_GW,), lambda i: (i,)),
            ],
            core_axis_name=("c", "s"),
            dimension_semantics=(pltpu.PARALLEL,),
        )(i_hbm, osec_hbm, ofst_hbm)

    return k(*sec_es, fst16, idx)


def _tc_block(f, e, sec_ref, fst_ref, xv_ref, w1t_ref, b1_ref, w2t_ref,
              b2_ref, bias_ref, out_ref):
    hi = lax.Precision.HIGHEST
    sec_raw = sec_ref[...]
    xv = xv_ref[...]

    li = lax.broadcasted_iota(jnp.int32, (f * e, f), 0)
    fi = lax.broadcasted_iota(jnp.int32, (f * e, f), 1)
    erep = (li // e == fi).astype(jnp.float32)
    sec = sec_raw * jnp.dot(erep, xv, precision=hi)

    g1 = lax.broadcasted_iota(jnp.int32, (e, f * e), 0)
    g2 = lax.broadcasted_iota(jnp.int32, (e, f * e), 1)
    grep = (g2 % e == g1).astype(jnp.float32)
    s1 = jnp.dot(grep, sec, precision=hi)
    s2 = jnp.dot(grep, sec * sec, precision=hi)
    fm = 0.5 * (s1 * s1 - s2)

    h = jnp.maximum(jnp.dot(w1t_ref[...], sec, precision=hi) + b1_ref[...], 0.0)
    d = jnp.maximum(jnp.dot(w2t_ref[...], h, precision=hi) + b2_ref[...], 0.0)

    fst_sum = jnp.sum(fst_ref[...] * xv, axis=0, keepdims=True)
    out_ref[...] = (fst_sum + jnp.sum(fm, axis=0, keepdims=True)
                    + jnp.sum(d, axis=0, keepdims=True) + bias_ref[...])


def kernel(Xi, Xv, fst_tables, sec_tables, W1, b1, W2, b2, bias):
    b_sz, f, _ = Xi.shape
    v = sec_tables.shape[1]
    e = sec_tables.shape[2]
    h1 = W1.shape[1]
    h2 = W2.shape[1]

    sec_t3 = jnp.transpose(sec_tables, (0, 2, 1))
    fst_t2 = jnp.transpose(fst_tables, (0, 2, 1))
    tabs = _split_repack(sec_t3, fst_t2, f, v, e)
    nrow = f * _VP // 16
    sec_es = [t.reshape(nrow, 16) for t in tabs[:e]]
    fst16 = tabs[e].reshape(nrow, 16)

    xi_t = Xi[:, :, 0].astype(jnp.int32).T
    offs = jnp.arange(f, dtype=jnp.int32)[:, None]
    idx = (xi_t + offs * _VP).reshape(b_sz * f)

    sec_g, fst_g = _sc_gather(sec_es, fst16, idx, f, e, b_sz)
    sec_g = sec_g.reshape(f * e, b_sz)
    fst_g = fst_g.reshape(f, b_sz)
    xvt = Xv[:, :, 0].T

    bt = 2048
    out = pl.pallas_call(
        functools.partial(_tc_block, f, e),
        grid=(b_sz // bt,),
        in_specs=[
            pl.BlockSpec((f * e, bt), lambda i: (0, i)),
            pl.BlockSpec((f, bt), lambda i: (0, i)),
            pl.BlockSpec((f, bt), lambda i: (0, i)),
            pl.BlockSpec((h1, f * e), lambda i: (0, 0)),
            pl.BlockSpec((h1, 1), lambda i: (0, 0)),
            pl.BlockSpec((h2, h1), lambda i: (0, 0)),
            pl.BlockSpec((h2, 1), lambda i: (0, 0)),
            pl.BlockSpec((1, 1), lambda i: (0, 0)),
        ],
        out_specs=pl.BlockSpec((1, bt), lambda i: (0, i)),
        out_shape=jax.ShapeDtypeStruct((1, b_sz), jnp.float32),
    )(sec_g, fst_g, xvt, W1.T, b1.reshape(h1, 1), W2.T, b2.reshape(h2, 1),
      bias.reshape(1, 1))
    return out.reshape(b_sz)
